# Initial kernel scaffold; baseline (speedup 1.0000x reference)
import jax
import jax.numpy as jnp
from jax.experimental import pallas as pl


def kernel(patch, patch_lib):
    raise NotImplementedError("write your pallas kernel here")


# speedup vs baseline: 1.2677x; 1.2677x over previous
import functools

import jax
import jax.numpy as jnp
from jax import lax
from jax.experimental import pallas as pl
from jax.experimental.pallas import tpu as pltpu

Q, D = 3136, 1152
K = 16384
KB = 256
NKB = K // KB
FEAT = 56
IMG = 224
BIG_I32 = 2**30
F32_INF = float("inf")


def _k1_body(p_ref, lib_ref, sq_ref, mt_ref, j_ref,
             q2_ref, acc_ref, blk_ref):
    k = pl.program_id(0)
    p = p_ref[...]
    b = lib_ref[...]

    @pl.when(k == 0)
    def _init():
        q2_ref[...] = jnp.sum(p * p, axis=1, keepdims=True)
        acc_ref[...] = jnp.full((Q, KB), F32_INF, jnp.float32)
        blk_ref[...] = jnp.zeros((Q, KB), jnp.int32)

    b2 = jnp.sum(b * b, axis=1)[None, :]
    dot = lax.dot_general(p, b, (((1,), (1,)), ((), ())),
                          preferred_element_type=jnp.float32)
    sq = q2_ref[...] + b2 - 2.0 * dot
    acc = acc_ref[...]
    upd = sq < acc
    acc_ref[...] = jnp.where(upd, sq, acc)
    blk_ref[...] = jnp.where(upd, k, blk_ref[...])

    @pl.when(k == NKB - 1)
    def _final():
        accf = acc_ref[...]
        sqmin = jnp.maximum(jnp.min(accf, axis=1, keepdims=True), 0.0)
        sq_ref[...] = sqmin
        s_val = jnp.max(sqmin, axis=(0, 1), keepdims=True)
        rid = lax.broadcasted_iota(jnp.int32, (Q, 1), 0)
        s_idx = jnp.min(jnp.where(sqmin == s_val, rid, BIG_I32),
                        axis=(0, 1), keepdims=True)
        rmask = rid == s_idx
        mt_ref[...] = jnp.sum(jnp.where(rmask, p, 0.0), axis=0,
                              keepdims=True)
        vrow = jnp.sum(jnp.where(rmask, accf, 0.0), axis=0,
                       keepdims=True)
        brow = jnp.sum(jnp.where(rmask, blk_ref[...], 0), axis=0,
                       keepdims=True)
        mval = jnp.min(vrow, axis=1, keepdims=True)
        lit = lax.broadcasted_iota(jnp.int32, (1, KB), 1)
        lidx = jnp.min(jnp.where(vrow == mval, lit, BIG_I32), axis=1,
                       keepdims=True)
        bsel = jnp.sum(jnp.where(lit == lidx, brow, 0), axis=1,
                       keepdims=True)
        j_ref[...] = bsel * KB + lidx


def _k2_body(lhs_ref, lib_ref, min56_ref, r_ref, rt_ref,
             s_ref, map_ref, rows_ref):
    k = pl.program_id(0)
    lhs = lhs_ref[...]
    b = lib_ref[...]
    l2 = jnp.sum(lhs * lhs, axis=1, keepdims=True)
    b2 = jnp.sum(b * b, axis=1)[None, :]
    dot = lax.dot_general(lhs, b, (((1,), (1,)), ((), ())),
                          preferred_element_type=jnp.float32)
    rows_ref[:, pl.ds(k * KB, KB)] = l2 + b2 - 2.0 * dot

    @pl.when(k == NKB - 1)
    def _final():
        w = jnp.maximum(rows_ref[0:1, :], 0.0)
        t = rows_ref[1:2, :]
        it = lax.broadcasted_iota(jnp.int32, (1, K), 1)

        def take_min(wv):
            m = jnp.min(wv, axis=1, keepdims=True)
            i = jnp.min(jnp.where(wv == m, it, BIG_I32), axis=1,
                        keepdims=True)
            return i, jnp.where(it == i, F32_INF, wv)

        i1, w1 = take_min(w)
        i2, w2 = take_min(w1)
        i3, _ = take_min(w2)
        t2 = jnp.sum(jnp.where(it == i2, t, 0.0), axis=1, keepdims=True)
        t3 = jnp.sum(jnp.where(it == i3, t, 0.0), axis=1, keepdims=True)
        knn2 = jnp.sqrt(jnp.maximum(t2, 0.0))
        knn3 = jnp.sqrt(jnp.maximum(t3, 0.0))
        min56 = min56_ref[...]
        s_sq = jnp.max(min56, axis=(0, 1), keepdims=True)
        s_star = jnp.sqrt(s_sq)
        dsqrt = jnp.sqrt(jnp.float32(D))
        wcoef = 1.0 - jnp.exp(s_star / dsqrt) / (
            jnp.exp(knn2 / dsqrt) + jnp.exp(knn3 / dsqrt))
        s_ref[...] = wcoef * s_star
        d56 = jnp.sqrt(min56)
        tmp = lax.dot_general(r_ref[...], d56, (((1,), (0,)), ((), ())),
                              preferred_element_type=jnp.float32)
        map_ref[...] = lax.dot_general(tmp, rt_ref[...],
                                       (((1,), (0,)), ((), ())),
                                       preferred_element_type=jnp.float32)


@functools.partial(jax.jit, static_argnames=())
def kernel(patch, patch_lib):
    patch = patch.astype(jnp.float32)
    patch_lib = patch_lib.astype(jnp.float32)

    sqmin, m_test, jstar = pl.pallas_call(
        _k1_body,
        grid=(NKB,),
        in_specs=[
            pl.BlockSpec((Q, D), lambda k: (0, 0)),
            pl.BlockSpec((KB, D), lambda k: (k, 0)),
        ],
        out_specs=[
            pl.BlockSpec((Q, 1), lambda k: (0, 0)),
            pl.BlockSpec((1, D), lambda k: (0, 0)),
            pl.BlockSpec((1, 1), lambda k: (0, 0)),
        ],
        out_shape=[
            jax.ShapeDtypeStruct((Q, 1), jnp.float32),
            jax.ShapeDtypeStruct((1, D), jnp.float32),
            jax.ShapeDtypeStruct((1, 1), jnp.int32),
        ],
        scratch_shapes=[
            pltpu.VMEM((Q, 1), jnp.float32),
            pltpu.VMEM((Q, KB), jnp.float32),
            pltpu.VMEM((Q, KB), jnp.int32),
        ],
    )(patch, patch_lib)

    m_star = lax.dynamic_slice(patch_lib, (jstar[0, 0], 0), (1, D))
    lhs = jnp.concatenate([m_star, m_test], axis=0)
    min56 = sqmin.reshape(FEAT, FEAT)
    r_mat = jax.image.resize(jnp.eye(FEAT, dtype=jnp.float32),
                             (IMG, FEAT), method="bilinear")

    s, s_map = pl.pallas_call(
        _k2_body,
        grid=(NKB,),
        in_specs=[
            pl.BlockSpec((2, D), lambda k: (0, 0)),
            pl.BlockSpec((KB, D), lambda k: (k, 0)),
            pl.BlockSpec((FEAT, FEAT), lambda k: (0, 0)),
            pl.BlockSpec((IMG, FEAT), lambda k: (0, 0)),
            pl.BlockSpec((FEAT, IMG), lambda k: (0, 0)),
        ],
        out_specs=[
            pl.BlockSpec((1, 1), lambda k: (0, 0)),
            pl.BlockSpec((IMG, IMG), lambda k: (0, 0)),
        ],
        out_shape=[
            jax.ShapeDtypeStruct((1, 1), jnp.float32),
            jax.ShapeDtypeStruct((IMG, IMG), jnp.float32),
        ],
        scratch_shapes=[
            pltpu.VMEM((2, K), jnp.float32),
        ],
    )(lhs, patch_lib, min56, r_mat, r_mat.T)

    return (s[0, 0], s_map.reshape(1, 1, IMG, IMG))


# speedup vs baseline: 1.3661x; 1.0776x over previous
import functools

import jax
import jax.numpy as jnp
from jax import lax
from jax.experimental import pallas as pl
from jax.experimental.pallas import tpu as pltpu

Q, D = 3136, 1152
K = 16384
KB = 512
NKB = K // KB
FEAT = 56
IMG = 224
BIG_I32 = 2**30
F32_INF = float("inf")


def _k1_body(p_ref, lib_ref, sq_ref, mt_ref, q2_ref, acc_ref):
    k = pl.program_id(0)
    p = p_ref[...]
    b = lib_ref[...]

    @pl.when(k == 0)
    def _init():
        q2_ref[...] = jnp.sum(p * p, axis=1, keepdims=True)
        acc_ref[...] = jnp.full((Q, KB), F32_INF, jnp.float32)

    b2 = jnp.sum(b * b, axis=1)[None, :]
    dot = lax.dot_general(p, b, (((1,), (1,)), ((), ())),
                          preferred_element_type=jnp.float32)
    sq = q2_ref[...] + b2 - 2.0 * dot
    acc = acc_ref[...]
    acc_ref[...] = jnp.where(sq < acc, sq, acc)

    @pl.when(k == NKB - 1)
    def _final():
        sqmin = jnp.maximum(jnp.min(acc_ref[...], axis=1, keepdims=True),
                            0.0)
        sq_ref[...] = sqmin
        s_val = jnp.max(sqmin, axis=(0, 1), keepdims=True)
        rid = lax.broadcasted_iota(jnp.int32, (Q, 1), 0)
        s_idx = jnp.min(jnp.where(sqmin == s_val, rid, BIG_I32),
                        axis=(0, 1), keepdims=True)
        rmask = rid == s_idx
        mt_ref[...] = jnp.sum(jnp.where(rmask, p, 0.0), axis=0,
                              keepdims=True)


def _k2a_body(mt_ref, lib_ref, trow_ref, j_ref, scr_ref):
    k = pl.program_id(0)
    m = mt_ref[...]
    b = lib_ref[...]
    l2 = jnp.sum(m * m, axis=1, keepdims=True)
    b2 = jnp.sum(b * b, axis=1)[None, :]
    dot = lax.dot_general(m, b, (((1,), (1,)), ((), ())),
                          preferred_element_type=jnp.float32)
    scr_ref[:, pl.ds(k * KB, KB)] = l2 + b2 - 2.0 * dot

    @pl.when(k == NKB - 1)
    def _final():
        t = scr_ref[...]
        trow_ref[...] = t
        mval = jnp.min(t, axis=1, keepdims=True)
        it = lax.broadcasted_iota(jnp.int32, (1, K), 1)
        j_ref[...] = jnp.min(jnp.where(t == mval, it, BIG_I32), axis=1,
                             keepdims=True)


def _k2b_body(ms_ref, lib_ref, trow_ref, min56_ref, r_ref, rt_ref,
              s_ref, map_ref, scr_ref):
    k = pl.program_id(0)
    m = ms_ref[...]
    b = lib_ref[...]
    l2 = jnp.sum(m * m, axis=1, keepdims=True)
    b2 = jnp.sum(b * b, axis=1)[None, :]
    dot = lax.dot_general(m, b, (((1,), (1,)), ((), ())),
                          preferred_element_type=jnp.float32)
    scr_ref[:, pl.ds(k * KB, KB)] = l2 + b2 - 2.0 * dot

    @pl.when(k == NKB - 1)
    def _final():
        w = jnp.maximum(scr_ref[...], 0.0)
        t = trow_ref[...]
        it = lax.broadcasted_iota(jnp.int32, (1, K), 1)

        def take_min(wv):
            mv = jnp.min(wv, axis=1, keepdims=True)
            i = jnp.min(jnp.where(wv == mv, it, BIG_I32), axis=1,
                        keepdims=True)
            return i, jnp.where(it == i, F32_INF, wv)

        i1, w1 = take_min(w)
        i2, w2 = take_min(w1)
        i3, _ = take_min(w2)
        t2 = jnp.sum(jnp.where(it == i2, t, 0.0), axis=1, keepdims=True)
        t3 = jnp.sum(jnp.where(it == i3, t, 0.0), axis=1, keepdims=True)
        knn2 = jnp.sqrt(jnp.maximum(t2, 0.0))
        knn3 = jnp.sqrt(jnp.maximum(t3, 0.0))
        min56 = min56_ref[...]
        s_sq = jnp.max(min56, axis=(0, 1), keepdims=True)
        s_star = jnp.sqrt(s_sq)
        dsqrt = jnp.sqrt(jnp.float32(D))
        wcoef = 1.0 - jnp.exp(s_star / dsqrt) / (
            jnp.exp(knn2 / dsqrt) + jnp.exp(knn3 / dsqrt))
        s_ref[...] = wcoef * s_star
        d56 = jnp.sqrt(min56)
        tmp = lax.dot_general(r_ref[...], d56, (((1,), (0,)), ((), ())),
                              preferred_element_type=jnp.float32)
        map_ref[...] = lax.dot_general(tmp, rt_ref[...],
                                       (((1,), (0,)), ((), ())),
                                       preferred_element_type=jnp.float32)


@functools.partial(jax.jit, static_argnames=())
def kernel(patch, patch_lib):
    patch = patch.astype(jnp.float32)
    patch_lib = patch_lib.astype(jnp.float32)

    sqmin, m_test = pl.pallas_call(
        _k1_body,
        grid=(NKB,),
        in_specs=[
            pl.BlockSpec((Q, D), lambda k: (0, 0)),
            pl.BlockSpec((KB, D), lambda k: (k, 0)),
        ],
        out_specs=[
            pl.BlockSpec((Q, 1), lambda k: (0, 0)),
            pl.BlockSpec((1, D), lambda k: (0, 0)),
        ],
        out_shape=[
            jax.ShapeDtypeStruct((Q, 1), jnp.float32),
            jax.ShapeDtypeStruct((1, D), jnp.float32),
        ],
        scratch_shapes=[
            pltpu.VMEM((Q, 1), jnp.float32),
            pltpu.VMEM((Q, KB), jnp.float32),
        ],
    )(patch, patch_lib)

    trow, jstar = pl.pallas_call(
        _k2a_body,
        grid=(NKB,),
        in_specs=[
            pl.BlockSpec((1, D), lambda k: (0, 0)),
            pl.BlockSpec((KB, D), lambda k: (k, 0)),
        ],
        out_specs=[
            pl.BlockSpec((1, K), lambda k: (0, 0)),
            pl.BlockSpec((1, 1), lambda k: (0, 0)),
        ],
        out_shape=[
            jax.ShapeDtypeStruct((1, K), jnp.float32),
            jax.ShapeDtypeStruct((1, 1), jnp.int32),
        ],
        scratch_shapes=[
            pltpu.VMEM((1, K), jnp.float32),
        ],
    )(m_test, patch_lib)

    m_star = lax.dynamic_slice(patch_lib, (jstar[0, 0], 0), (1, D))
    min56 = sqmin.reshape(FEAT, FEAT)
    r_mat = jax.image.resize(jnp.eye(FEAT, dtype=jnp.float32),
                             (IMG, FEAT), method="bilinear")

    s, s_map = pl.pallas_call(
        _k2b_body,
        grid=(NKB,),
        in_specs=[
            pl.BlockSpec((1, D), lambda k: (0, 0)),
            pl.BlockSpec((KB, D), lambda k: (k, 0)),
            pl.BlockSpec((1, K), lambda k: (0, 0)),
            pl.BlockSpec((FEAT, FEAT), lambda k: (0, 0)),
            pl.BlockSpec((IMG, FEAT), lambda k: (0, 0)),
            pl.BlockSpec((FEAT, IMG), lambda k: (0, 0)),
        ],
        out_specs=[
            pl.BlockSpec((1, 1), lambda k: (0, 0)),
            pl.BlockSpec((IMG, IMG), lambda k: (0, 0)),
        ],
        out_shape=[
            jax.ShapeDtypeStruct((1, 1), jnp.float32),
            jax.ShapeDtypeStruct((IMG, IMG), jnp.float32),
        ],
        scratch_shapes=[
            pltpu.VMEM((1, K), jnp.float32),
        ],
    )(m_star, patch_lib, trow, min56, r_mat, r_mat.T)

    return (s[0, 0], s_map.reshape(1, 1, IMG, IMG))
